# baseline (speedup 1.0000x reference)
import jax
import jax.numpy as jnp
from jax import lax
from jax.experimental import pallas as pl
from jax.experimental.pallas import tpu as pltpu
from jax.experimental.pallas import tpu_sc as plsc

L, B, T, H, D = 8, 8, 64, 8, 64
S = 512
LB = L * B
HD = H * D
NC, NS = 2, 16
NW = NC * NS
ROWS_PER_W = LB // NW
ZPT = S // NS

_SC_PARAMS = pltpu.CompilerParams(use_tc_tiling_on_sc=True)
_MESH = plsc.VectorSubcoreMesh(core_axis_name="c", subcore_axis_name="s")


def _sc_zero_body(zb_hbm, zbuf, zspmem, sem):
    c = lax.axis_index("c")
    s = lax.axis_index("s")
    wid = s * NC + c
    zero16 = jnp.zeros((16,), jnp.float32)

    def zfill(t, carry):
        for q in range(HD // 16):
            zbuf[t, pl.ds(q * 16, 16)] = zero16
        return carry

    lax.fori_loop(0, ZPT, zfill, 0)
    off = pl.multiple_of(s * ZPT, ZPT)
    pltpu.sync_copy(zbuf, zspmem.at[pl.ds(off, ZPT)])
    plsc.subcore_barrier()

    copies = []
    for rl in range(ROWS_PER_W):
        r = wid * ROWS_PER_W + rl
        copies.append(pltpu.async_copy(zspmem, zb_hbm.at[r], sem))
    for cp in copies:
        cp.wait()


_sc_zero = pl.kernel(
    _sc_zero_body,
    out_type=jax.ShapeDtypeStruct((LB, HD, S), jnp.float32),
    mesh=_MESH,
    scratch_types=[
        pltpu.VMEM((ZPT, S), jnp.float32),
        pltpu.VMEM_SHARED((HD, S), jnp.float32),
        pltpu.SemaphoreType.DMA,
    ],
    compiler_params=_SC_PARAMS,
)


def _sc_copy_body(k_hbm, v_hbm, ko_hbm, vo_hbm, stage, sem):
    c = lax.axis_index("c")
    s = lax.axis_index("s")
    wid = s * NC + c
    for rl in range(ROWS_PER_W):
        r = wid * ROWS_PER_W + rl
        li = r // B
        bi = r % B
        pltpu.async_copy(k_hbm.at[li, bi], stage, sem).wait()
        pltpu.async_copy(stage, ko_hbm.at[li, bi], sem).wait()
        pltpu.async_copy(v_hbm.at[li, bi], stage, sem).wait()
        pltpu.async_copy(stage, vo_hbm.at[li, bi], sem).wait()


_sc_copy = pl.kernel(
    _sc_copy_body,
    out_type=[
        jax.ShapeDtypeStruct((L, B, T, H, D), jnp.float32),
        jax.ShapeDtypeStruct((L, B, T, H, D), jnp.float32),
    ],
    mesh=_MESH,
    scratch_types=[
        pltpu.VMEM((T, H, D), jnp.float32),
        pltpu.SemaphoreType.DMA,
    ],
    compiler_params=_SC_PARAMS,
)


def _tc_tok_body(k_ref, z_ref, o_ref):
    k = k_ref[0, 0]
    o_ref[0, :, T:] = jnp.zeros((HD, T), jnp.float32)
    for h in range(H):
        o_ref[0, pl.ds(h * D, D), :T] = jnp.transpose(k[:, h, :])


def _tc_tok(src5, zbuf3):
    in5 = pl.BlockSpec((1, 1, T, H, D), lambda i: (i // B, i % B, 0, 0, 0))
    tok = pl.BlockSpec((1, HD, 2 * T), lambda i: (i, 0, 0))
    return pl.pallas_call(
        _tc_tok_body,
        grid=(LB,),
        in_specs=[in5, tok],
        out_specs=tok,
        out_shape=jax.ShapeDtypeStruct((LB, HD, S), jnp.float32),
        input_output_aliases={1: 0},
    )(src5, zbuf3)


def kernel(keys, values, keys_buf, values_buf):
    kbz = _sc_zero()
    vbz = _sc_zero()
    ko, vo = _sc_copy(keys, values)
    kbp = _tc_tok(keys, kbz)
    vbp = _tc_tok(values, vbz)
    kb = jnp.transpose(kbp.reshape(L, B, H, D, S), (0, 1, 4, 2, 3))
    vb = jnp.transpose(vbp.reshape(L, B, H, D, S), (0, 1, 4, 2, 3))
    return (kb, vb, ko, vo)


# speedup vs baseline: 1.1733x; 1.1733x over previous
import jax
import jax.numpy as jnp
from jax import lax
from jax.experimental import pallas as pl
from jax.experimental.pallas import tpu as pltpu
from jax.experimental.pallas import tpu_sc as plsc

L, B, T, H, D = 8, 8, 64, 8, 64
S = 512
LB = L * B
HD = H * D
NC, NS = 2, 16
NW = NC * NS
ROWS_PER_W = LB // NW
SZ = S - 2 * T
ZPT = HD // NS

_SC_PARAMS = pltpu.CompilerParams(use_tc_tiling_on_sc=True)
_MESH = plsc.VectorSubcoreMesh(core_axis_name="c", subcore_axis_name="s")


def _sc_zero_body(kb_hbm, vb_hbm, zbuf, zspmem, sem):
    c = lax.axis_index("c")
    s = lax.axis_index("s")
    wid = s * NC + c
    zero16 = jnp.zeros((16,), jnp.float32)

    def zfill(t, carry):
        for q in range(SZ // 16):
            zbuf[t, pl.ds(q * 16, 16)] = zero16
        return carry

    lax.fori_loop(0, ZPT, zfill, 0)
    off = pl.multiple_of(s * ZPT, ZPT)
    pltpu.sync_copy(zbuf, zspmem.at[pl.ds(off, ZPT)])
    plsc.subcore_barrier()

    copies = []
    for rl in range(ROWS_PER_W):
        r = wid * ROWS_PER_W + rl
        copies.append(pltpu.async_copy(
            zspmem, kb_hbm.at[r, :, pl.ds(2 * T, SZ)], sem))
        copies.append(pltpu.async_copy(
            zspmem, vb_hbm.at[r, :, pl.ds(2 * T, SZ)], sem))
    for cp in copies:
        cp.wait()


_sc_zero = pl.kernel(
    _sc_zero_body,
    out_type=[
        jax.ShapeDtypeStruct((LB, HD, S), jnp.float32),
        jax.ShapeDtypeStruct((LB, HD, S), jnp.float32),
    ],
    mesh=_MESH,
    scratch_types=[
        pltpu.VMEM((ZPT, SZ), jnp.float32),
        pltpu.VMEM_SHARED((HD, SZ), jnp.float32),
        pltpu.SemaphoreType.DMA,
    ],
    compiler_params=_SC_PARAMS,
)


def _sc_copy_body(k_hbm, v_hbm, ko_hbm, vo_hbm, stage, sem):
    c = lax.axis_index("c")
    s = lax.axis_index("s")
    wid = s * NC + c
    for rl in range(ROWS_PER_W):
        r = wid * ROWS_PER_W + rl
        li = r // B
        bi = r % B
        pltpu.async_copy(k_hbm.at[li, bi], stage, sem).wait()
        pltpu.async_copy(stage, ko_hbm.at[li, bi], sem).wait()
        pltpu.async_copy(v_hbm.at[li, bi], stage, sem).wait()
        pltpu.async_copy(stage, vo_hbm.at[li, bi], sem).wait()


_sc_copy = pl.kernel(
    _sc_copy_body,
    out_type=[
        jax.ShapeDtypeStruct((L, B, T, H, D), jnp.float32),
        jax.ShapeDtypeStruct((L, B, T, H, D), jnp.float32),
    ],
    mesh=_MESH,
    scratch_types=[
        pltpu.VMEM((T, H, D), jnp.float32),
        pltpu.SemaphoreType.DMA,
    ],
    compiler_params=_SC_PARAMS,
)


def _tc_tok_body(k_ref, v_ref, kbz_ref, vbz_ref, kb_ref, vb_ref):
    k = k_ref[0, 0]
    v = v_ref[0, 0]
    zpad = jnp.zeros((HD, T), jnp.float32)
    kb_ref[0, :, T:] = zpad
    vb_ref[0, :, T:] = zpad
    for h in range(H):
        kb_ref[0, pl.ds(h * D, D), :T] = jnp.transpose(k[:, h, :])
        vb_ref[0, pl.ds(h * D, D), :T] = jnp.transpose(v[:, h, :])


def _tc_tok(keys, values, kbz, vbz):
    in5 = pl.BlockSpec((1, 1, T, H, D), lambda i: (i // B, i % B, 0, 0, 0))
    tok = pl.BlockSpec((1, HD, 2 * T), lambda i: (i, 0, 0))
    return pl.pallas_call(
        _tc_tok_body,
        grid=(LB,),
        in_specs=[in5, in5, tok, tok],
        out_specs=[tok, tok],
        out_shape=[
            jax.ShapeDtypeStruct((LB, HD, S), jnp.float32),
            jax.ShapeDtypeStruct((LB, HD, S), jnp.float32),
        ],
        input_output_aliases={2: 0, 3: 1},
    )(keys, values, kbz, vbz)


def kernel(keys, values, keys_buf, values_buf):
    kbz, vbz = _sc_zero()
    ko, vo = _sc_copy(keys, values)
    kbp, vbp = _tc_tok(keys, values, kbz, vbz)
    kb = jnp.transpose(kbp.reshape(L, B, H, D, S), (0, 1, 4, 2, 3))
    vb = jnp.transpose(vbp.reshape(L, B, H, D, S), (0, 1, 4, 2, 3))
    return (kb, vb, ko, vo)


# speedup vs baseline: 1.1956x; 1.0190x over previous
import jax
import jax.numpy as jnp
from jax import lax
from jax.experimental import pallas as pl
from jax.experimental.pallas import tpu as pltpu
from jax.experimental.pallas import tpu_sc as plsc

L, B, T, H, D = 8, 8, 64, 8, 64
S = 512
LB = L * B
HD = H * D
NC, NS = 2, 16
NW = NC * NS
ROWS_PER_W = LB // NW
SZ = S - 2 * T
ZPT = HD // NS

_SC_PARAMS = pltpu.CompilerParams(use_tc_tiling_on_sc=True)
_MESH = plsc.VectorSubcoreMesh(core_axis_name="c", subcore_axis_name="s")


def _sc_zero_body(kb_hbm, vb_hbm, zbuf, zspmem, sem):
    c = lax.axis_index("c")
    s = lax.axis_index("s")
    wid = s * NC + c
    zero16 = jnp.zeros((16,), jnp.float32)

    def zfill(t, carry):
        for q in range(SZ // 16):
            zbuf[t, pl.ds(q * 16, 16)] = zero16
        return carry

    lax.fori_loop(0, ZPT, zfill, 0)
    off = pl.multiple_of(s * ZPT, ZPT)
    pltpu.sync_copy(zbuf, zspmem.at[pl.ds(off, ZPT)])
    plsc.subcore_barrier()

    copies = []
    for rl in range(ROWS_PER_W):
        r = wid * ROWS_PER_W + rl
        copies.append(pltpu.async_copy(
            zspmem, kb_hbm.at[r, :, pl.ds(2 * T, SZ)], sem))
        copies.append(pltpu.async_copy(
            zspmem, vb_hbm.at[r, :, pl.ds(2 * T, SZ)], sem))
    for cp in copies:
        cp.wait()


_sc_zero = pl.kernel(
    _sc_zero_body,
    out_type=[
        jax.ShapeDtypeStruct((LB, HD, S), jnp.float32),
        jax.ShapeDtypeStruct((LB, HD, S), jnp.float32),
    ],
    mesh=_MESH,
    scratch_types=[
        pltpu.VMEM((ZPT, SZ), jnp.float32),
        pltpu.VMEM_SHARED((HD, SZ), jnp.float32),
        pltpu.SemaphoreType.DMA,
    ],
    compiler_params=_SC_PARAMS,
)


def _sc_copy_body(k_hbm, v_hbm, ko_hbm, vo_hbm, stage, sem):
    c = lax.axis_index("c")
    s = lax.axis_index("s")
    wid = s * NC + c
    for rl in range(ROWS_PER_W):
        r = wid * ROWS_PER_W + rl
        li = r // B
        bi = r % B
        pltpu.async_copy(k_hbm.at[li, bi], stage, sem).wait()
        pltpu.async_copy(stage, ko_hbm.at[li, bi], sem).wait()
        pltpu.async_copy(v_hbm.at[li, bi], stage, sem).wait()
        pltpu.async_copy(stage, vo_hbm.at[li, bi], sem).wait()


_sc_copy = pl.kernel(
    _sc_copy_body,
    out_type=[
        jax.ShapeDtypeStruct((L, B, T, H, D), jnp.float32),
        jax.ShapeDtypeStruct((L, B, T, H, D), jnp.float32),
    ],
    mesh=_MESH,
    scratch_types=[
        pltpu.VMEM((T, H, D), jnp.float32),
        pltpu.SemaphoreType.DMA,
    ],
    compiler_params=_SC_PARAMS,
)


def _tc_tok_body(k_ref, v_ref, kbz_ref, vbz_ref, kb_ref, vb_ref, ko_ref, vo_ref):
    k = k_ref[...]
    v = v_ref[...]
    ko_ref[...] = k
    vo_ref[...] = v
    k = k[0, 0]
    v = v[0, 0]
    zpad = jnp.zeros((HD, T), jnp.float32)
    kb_ref[0, :, T:] = zpad
    vb_ref[0, :, T:] = zpad
    for h in range(H):
        kb_ref[0, pl.ds(h * D, D), :T] = jnp.transpose(k[:, h, :])
        vb_ref[0, pl.ds(h * D, D), :T] = jnp.transpose(v[:, h, :])


def _tc_tok(keys, values, kbz, vbz):
    in5 = pl.BlockSpec((1, 1, T, H, D), lambda i: (i // B, i % B, 0, 0, 0))
    tok = pl.BlockSpec((1, HD, 2 * T), lambda i: (i, 0, 0))
    return pl.pallas_call(
        _tc_tok_body,
        grid=(LB,),
        in_specs=[in5, in5, tok, tok],
        out_specs=[tok, tok, in5, in5],
        out_shape=[
            jax.ShapeDtypeStruct((LB, HD, S), jnp.float32),
            jax.ShapeDtypeStruct((LB, HD, S), jnp.float32),
            jax.ShapeDtypeStruct((L, B, T, H, D), jnp.float32),
            jax.ShapeDtypeStruct((L, B, T, H, D), jnp.float32),
        ],
        input_output_aliases={2: 0, 3: 1},
    )(keys, values, kbz, vbz)


def kernel(keys, values, keys_buf, values_buf):
    kbz, vbz = _sc_zero()
    kbp, vbp, ko, vo = _tc_tok(keys, values, kbz, vbz)
    kb = jnp.transpose(kbp.reshape(L, B, H, D, S), (0, 1, 4, 2, 3))
    vb = jnp.transpose(vbp.reshape(L, B, H, D, S), (0, 1, 4, 2, 3))
    return (kb, vb, ko, vo)


# speedup vs baseline: 1.3954x; 1.1671x over previous
import jax
import jax.numpy as jnp
from jax import lax
from jax.experimental import pallas as pl
from jax.experimental.pallas import tpu as pltpu
from jax.experimental.pallas import tpu_sc as plsc

L, B, T, H, D = 8, 8, 64, 8, 64
S = 512
LB = L * B
HD = H * D
NC, NS = 2, 16
NW = NC * NS
ROWS_PER_W = LB // NW
SZ = S - 2 * T
ZPT = HD // NS

_SC_PARAMS = pltpu.CompilerParams(use_tc_tiling_on_sc=True)
_MESH = plsc.VectorSubcoreMesh(core_axis_name="c", subcore_axis_name="s")


def _sc_zero_body(kb_hbm, vb_hbm, zbuf, zspmem, sem):
    c = lax.axis_index("c")
    s = lax.axis_index("s")
    wid = s * NC + c
    zero16 = jnp.zeros((16,), jnp.float32)

    def zfill(t, carry):
        for q in range(SZ // 16):
            zbuf[t, pl.ds(q * 16, 16)] = zero16
        return carry

    lax.fori_loop(0, ZPT, zfill, 0)
    off = pl.multiple_of(s * ZPT, ZPT)
    pltpu.sync_copy(zbuf, zspmem.at[pl.ds(off, ZPT)])
    plsc.subcore_barrier()

    copies = []
    for rl in range(ROWS_PER_W):
        r = wid * ROWS_PER_W + rl
        copies.append(pltpu.async_copy(
            zspmem, kb_hbm.at[r, :, pl.ds(2 * T, SZ)], sem))
        copies.append(pltpu.async_copy(
            zspmem, vb_hbm.at[r, :, pl.ds(2 * T, SZ)], sem))
    for cp in copies:
        cp.wait()


_sc_zero = pl.kernel(
    _sc_zero_body,
    out_type=[
        jax.ShapeDtypeStruct((LB, HD, S), jnp.float32),
        jax.ShapeDtypeStruct((LB, HD, S), jnp.float32),
    ],
    mesh=_MESH,
    scratch_types=[
        pltpu.VMEM((ZPT, SZ), jnp.float32),
        pltpu.VMEM_SHARED((HD, SZ), jnp.float32),
        pltpu.SemaphoreType.DMA,
    ],
    compiler_params=_SC_PARAMS,
)


def _sc_copy_body(k_hbm, v_hbm, ko_hbm, vo_hbm, stage, sem):
    c = lax.axis_index("c")
    s = lax.axis_index("s")
    wid = s * NC + c
    for rl in range(ROWS_PER_W):
        r = wid * ROWS_PER_W + rl
        li = r // B
        bi = r % B
        pltpu.async_copy(k_hbm.at[li, bi], stage, sem).wait()
        pltpu.async_copy(stage, ko_hbm.at[li, bi], sem).wait()
        pltpu.async_copy(v_hbm.at[li, bi], stage, sem).wait()
        pltpu.async_copy(stage, vo_hbm.at[li, bi], sem).wait()


_sc_copy = pl.kernel(
    _sc_copy_body,
    out_type=[
        jax.ShapeDtypeStruct((L, B, T, H, D), jnp.float32),
        jax.ShapeDtypeStruct((L, B, T, H, D), jnp.float32),
    ],
    mesh=_MESH,
    scratch_types=[
        pltpu.VMEM((T, H, D), jnp.float32),
        pltpu.SemaphoreType.DMA,
    ],
    compiler_params=_SC_PARAMS,
)


_RPS = 2


def _tc_tok_body(k_ref, v_ref, kbz_ref, vbz_ref, kb_ref, vb_ref, ko_ref, vo_ref):
    k = k_ref[...]
    v = v_ref[...]
    ko_ref[...] = k
    vo_ref[...] = v
    zpad = jnp.zeros((HD, T), jnp.float32)
    for rb in range(_RPS):
        kb_ref[rb, :, T:] = zpad
        vb_ref[rb, :, T:] = zpad
        for h in range(H):
            kb_ref[rb, pl.ds(h * D, D), :T] = jnp.transpose(k[0, rb, :, h, :])
            vb_ref[rb, pl.ds(h * D, D), :T] = jnp.transpose(v[0, rb, :, h, :])


def _tc_tok(keys, values, kbz, vbz):
    bpg = B // _RPS
    in5 = pl.BlockSpec((1, _RPS, T, H, D),
                       lambda i: (i // bpg, i % bpg, 0, 0, 0))
    tok = pl.BlockSpec((_RPS, HD, 2 * T), lambda i: (i, 0, 0))
    return pl.pallas_call(
        _tc_tok_body,
        grid=(LB // _RPS,),
        in_specs=[in5, in5, tok, tok],
        out_specs=[tok, tok, in5, in5],
        out_shape=[
            jax.ShapeDtypeStruct((LB, HD, S), jnp.float32),
            jax.ShapeDtypeStruct((LB, HD, S), jnp.float32),
            jax.ShapeDtypeStruct((L, B, T, H, D), jnp.float32),
            jax.ShapeDtypeStruct((L, B, T, H, D), jnp.float32),
        ],
        input_output_aliases={2: 0, 3: 1},
    )(keys, values, kbz, vbz)


def kernel(keys, values, keys_buf, values_buf):
    kbz, vbz = _sc_zero()
    kbp, vbp, ko, vo = _tc_tok(keys, values, kbz, vbz)
    kb = jnp.transpose(kbp.reshape(L, B, H, D, S), (0, 1, 4, 2, 3))
    vb = jnp.transpose(vbp.reshape(L, B, H, D, S), (0, 1, 4, 2, 3))
    return (kb, vb, ko, vo)


# speedup vs baseline: 1.4724x; 1.0552x over previous
import jax
import jax.numpy as jnp
from jax import lax
from jax.experimental import pallas as pl
from jax.experimental.pallas import tpu as pltpu
from jax.experimental.pallas import tpu_sc as plsc

L, B, T, H, D = 8, 8, 64, 8, 64
S = 512
LB = L * B
HD = H * D
NC, NS = 2, 16
NW = NC * NS
ROWS_PER_W = LB // NW
SZ = S - 2 * T
ZPT = HD // NS

_SC_PARAMS = pltpu.CompilerParams(use_tc_tiling_on_sc=True)
_MESH = plsc.VectorSubcoreMesh(core_axis_name="c", subcore_axis_name="s")


def _sc_zero_body(kb_hbm, vb_hbm, zbuf, zspmem, sem):
    c = lax.axis_index("c")
    s = lax.axis_index("s")
    wid = s * NC + c
    zero16 = jnp.zeros((16,), jnp.float32)

    def zfill(t, carry):
        for q in range(SZ // 16):
            zbuf[t, pl.ds(q * 16, 16)] = zero16
        return carry

    lax.fori_loop(0, ZPT, zfill, 0)
    off = pl.multiple_of(s * ZPT, ZPT)
    pltpu.sync_copy(zbuf, zspmem.at[pl.ds(off, ZPT)])
    plsc.subcore_barrier()

    copies = []
    for rl in range(ROWS_PER_W):
        r = wid * ROWS_PER_W + rl
        copies.append(pltpu.async_copy(
            zspmem, kb_hbm.at[r, :, pl.ds(2 * T, SZ)], sem))
        copies.append(pltpu.async_copy(
            zspmem, vb_hbm.at[r, :, pl.ds(2 * T, SZ)], sem))
    for cp in copies:
        cp.wait()


_sc_zero = pl.kernel(
    _sc_zero_body,
    out_type=[
        jax.ShapeDtypeStruct((LB, HD, S), jnp.float32),
        jax.ShapeDtypeStruct((LB, HD, S), jnp.float32),
    ],
    mesh=_MESH,
    scratch_types=[
        pltpu.VMEM((ZPT, SZ), jnp.float32),
        pltpu.VMEM_SHARED((HD, SZ), jnp.float32),
        pltpu.SemaphoreType.DMA,
    ],
    compiler_params=_SC_PARAMS,
)


def _sc_copy_body(k_hbm, v_hbm, ko_hbm, vo_hbm, stage, sem):
    c = lax.axis_index("c")
    s = lax.axis_index("s")
    wid = s * NC + c
    for rl in range(ROWS_PER_W):
        r = wid * ROWS_PER_W + rl
        li = r // B
        bi = r % B
        pltpu.async_copy(k_hbm.at[li, bi], stage, sem).wait()
        pltpu.async_copy(stage, ko_hbm.at[li, bi], sem).wait()
        pltpu.async_copy(v_hbm.at[li, bi], stage, sem).wait()
        pltpu.async_copy(stage, vo_hbm.at[li, bi], sem).wait()


_sc_copy = pl.kernel(
    _sc_copy_body,
    out_type=[
        jax.ShapeDtypeStruct((L, B, T, H, D), jnp.float32),
        jax.ShapeDtypeStruct((L, B, T, H, D), jnp.float32),
    ],
    mesh=_MESH,
    scratch_types=[
        pltpu.VMEM((T, H, D), jnp.float32),
        pltpu.SemaphoreType.DMA,
    ],
    compiler_params=_SC_PARAMS,
)


_RPS = 4


def _tc_tok_body(k_ref, v_ref, kbz_ref, vbz_ref, kb_ref, vb_ref, ko_ref, vo_ref):
    k = k_ref[...]
    v = v_ref[...]
    ko_ref[...] = k
    vo_ref[...] = v
    zpad = jnp.zeros((HD, T), jnp.float32)
    for rb in range(_RPS):
        kb_ref[rb, :, T:] = zpad
        vb_ref[rb, :, T:] = zpad
        for h in range(H):
            kb_ref[rb, pl.ds(h * D, D), :T] = jnp.transpose(k[0, rb, :, h, :])
            vb_ref[rb, pl.ds(h * D, D), :T] = jnp.transpose(v[0, rb, :, h, :])


def _tc_tok(keys, values, kbz, vbz):
    bpg = B // _RPS
    in5 = pl.BlockSpec((1, _RPS, T, H, D),
                       lambda i: (i // bpg, i % bpg, 0, 0, 0))
    tok = pl.BlockSpec((_RPS, HD, 2 * T), lambda i: (i, 0, 0))
    return pl.pallas_call(
        _tc_tok_body,
        grid=(LB // _RPS,),
        in_specs=[in5, in5, tok, tok],
        out_specs=[tok, tok, in5, in5],
        out_shape=[
            jax.ShapeDtypeStruct((LB, HD, S), jnp.float32),
            jax.ShapeDtypeStruct((LB, HD, S), jnp.float32),
            jax.ShapeDtypeStruct((L, B, T, H, D), jnp.float32),
            jax.ShapeDtypeStruct((L, B, T, H, D), jnp.float32),
        ],
        input_output_aliases={2: 0, 3: 1},
    )(keys, values, kbz, vbz)


def kernel(keys, values, keys_buf, values_buf):
    kbz, vbz = _sc_zero()
    kbp, vbp, ko, vo = _tc_tok(keys, values, kbz, vbz)
    kb = jnp.transpose(kbp.reshape(L, B, H, D, S), (0, 1, 4, 2, 3))
    vb = jnp.transpose(vbp.reshape(L, B, H, D, S), (0, 1, 4, 2, 3))
    return (kb, vb, ko, vo)


# speedup vs baseline: 1.4923x; 1.0135x over previous
import jax
import jax.numpy as jnp
from jax import lax
from jax.experimental import pallas as pl
from jax.experimental.pallas import tpu as pltpu
from jax.experimental.pallas import tpu_sc as plsc

L, B, T, H, D = 8, 8, 64, 8, 64
S = 512
LB = L * B
HD = H * D
NC, NS = 2, 16
NW = NC * NS
ROWS_PER_W = LB // NW
SZ = S - 2 * T
ZPT = HD // NS

_SC_PARAMS = pltpu.CompilerParams(use_tc_tiling_on_sc=True)
_MESH = plsc.VectorSubcoreMesh(core_axis_name="c", subcore_axis_name="s")


def _sc_zero_body(kb_hbm, vb_hbm, zbuf, zspmem, sem):
    c = lax.axis_index("c")
    s = lax.axis_index("s")
    wid = s * NC + c
    zero16 = jnp.zeros((16,), jnp.float32)

    def zfill(t, carry):
        for q in range(SZ // 16):
            zbuf[t, pl.ds(q * 16, 16)] = zero16
        return carry

    lax.fori_loop(0, ZPT, zfill, 0)
    off = pl.multiple_of(s * ZPT, ZPT)
    pltpu.sync_copy(zbuf, zspmem.at[pl.ds(off, ZPT)])
    plsc.subcore_barrier()

    copies = []
    for rl in range(ROWS_PER_W):
        r = wid * ROWS_PER_W + rl
        copies.append(pltpu.async_copy(
            zspmem, kb_hbm.at[r, :, pl.ds(2 * T, SZ)], sem))
        copies.append(pltpu.async_copy(
            zspmem, vb_hbm.at[r, :, pl.ds(2 * T, SZ)], sem))
    for cp in copies:
        cp.wait()


_sc_zero = pl.kernel(
    _sc_zero_body,
    out_type=[
        jax.ShapeDtypeStruct((LB, HD, S), jnp.float32),
        jax.ShapeDtypeStruct((LB, HD, S), jnp.float32),
    ],
    mesh=_MESH,
    scratch_types=[
        pltpu.VMEM((ZPT, SZ), jnp.float32),
        pltpu.VMEM_SHARED((HD, SZ), jnp.float32),
        pltpu.SemaphoreType.DMA,
    ],
    compiler_params=_SC_PARAMS,
)


def _sc_copy_body(k_hbm, v_hbm, ko_hbm, vo_hbm, stage, sem):
    c = lax.axis_index("c")
    s = lax.axis_index("s")
    wid = s * NC + c
    for rl in range(ROWS_PER_W):
        r = wid * ROWS_PER_W + rl
        li = r // B
        bi = r % B
        pltpu.async_copy(k_hbm.at[li, bi], stage, sem).wait()
        pltpu.async_copy(stage, ko_hbm.at[li, bi], sem).wait()
        pltpu.async_copy(v_hbm.at[li, bi], stage, sem).wait()
        pltpu.async_copy(stage, vo_hbm.at[li, bi], sem).wait()


_sc_copy = pl.kernel(
    _sc_copy_body,
    out_type=[
        jax.ShapeDtypeStruct((L, B, T, H, D), jnp.float32),
        jax.ShapeDtypeStruct((L, B, T, H, D), jnp.float32),
    ],
    mesh=_MESH,
    scratch_types=[
        pltpu.VMEM((T, H, D), jnp.float32),
        pltpu.SemaphoreType.DMA,
    ],
    compiler_params=_SC_PARAMS,
)


_RPS = 8


def _tc_tok_body(k_ref, v_ref, kbz_ref, vbz_ref, kb_ref, vb_ref, ko_ref, vo_ref):
    k = k_ref[...]
    v = v_ref[...]
    ko_ref[...] = k
    vo_ref[...] = v
    zpad = jnp.zeros((HD, T), jnp.float32)
    for rb in range(_RPS):
        kb_ref[rb, :, T:] = zpad
        vb_ref[rb, :, T:] = zpad
        for h in range(H):
            kb_ref[rb, pl.ds(h * D, D), :T] = jnp.transpose(k[0, rb, :, h, :])
            vb_ref[rb, pl.ds(h * D, D), :T] = jnp.transpose(v[0, rb, :, h, :])


def _tc_tok(keys, values, kbz, vbz):
    bpg = B // _RPS
    in5 = pl.BlockSpec((1, _RPS, T, H, D),
                       lambda i: (i // bpg, i % bpg, 0, 0, 0))
    tok = pl.BlockSpec((_RPS, HD, 2 * T), lambda i: (i, 0, 0))
    return pl.pallas_call(
        _tc_tok_body,
        grid=(LB // _RPS,),
        in_specs=[in5, in5, tok, tok],
        out_specs=[tok, tok, in5, in5],
        out_shape=[
            jax.ShapeDtypeStruct((LB, HD, S), jnp.float32),
            jax.ShapeDtypeStruct((LB, HD, S), jnp.float32),
            jax.ShapeDtypeStruct((L, B, T, H, D), jnp.float32),
            jax.ShapeDtypeStruct((L, B, T, H, D), jnp.float32),
        ],
        input_output_aliases={2: 0, 3: 1},
    )(keys, values, kbz, vbz)


def kernel(keys, values, keys_buf, values_buf):
    kbz, vbz = _sc_zero()
    kbp, vbp, ko, vo = _tc_tok(keys, values, kbz, vbz)
    kb = jnp.transpose(kbp.reshape(L, B, H, D, S), (0, 1, 4, 2, 3))
    vb = jnp.transpose(vbp.reshape(L, B, H, D, S), (0, 1, 4, 2, 3))
    return (kb, vb, ko, vo)


# speedup vs baseline: 1.8776x; 1.2582x over previous
import jax
import jax.numpy as jnp
from jax import lax
from jax.experimental import pallas as pl
from jax.experimental.pallas import tpu as pltpu
from jax.experimental.pallas import tpu_sc as plsc

L, B, T, H, D = 8, 8, 64, 8, 64
S = 512
LB = L * B
HD = H * D
NC, NS = 2, 16
NW = NC * NS
ROWS_PER_W = LB // NW

_SC_PARAMS = pltpu.CompilerParams(use_tc_tiling_on_sc=True)
_MESH = plsc.VectorSubcoreMesh(core_axis_name="c", subcore_axis_name="s")


def _sc_copy_body(k_hbm, v_hbm, ko_hbm, vo_hbm, stage, sem):
    c = lax.axis_index("c")
    s = lax.axis_index("s")
    wid = s * NC + c
    for rl in range(ROWS_PER_W):
        r = wid * ROWS_PER_W + rl
        li = r // B
        bi = r % B
        pltpu.async_copy(k_hbm.at[li, bi], stage, sem).wait()
        pltpu.async_copy(stage, ko_hbm.at[li, bi], sem).wait()
        pltpu.async_copy(v_hbm.at[li, bi], stage, sem).wait()
        pltpu.async_copy(stage, vo_hbm.at[li, bi], sem).wait()


_sc_copy = pl.kernel(
    _sc_copy_body,
    out_type=[
        jax.ShapeDtypeStruct((L, B, T, H, D), jnp.float32),
        jax.ShapeDtypeStruct((L, B, T, H, D), jnp.float32),
    ],
    mesh=_MESH,
    scratch_types=[
        pltpu.VMEM((T, H, D), jnp.float32),
        pltpu.SemaphoreType.DMA,
    ],
    compiler_params=_SC_PARAMS,
)


_RPS = 4


def _tc_buf_body(k_ref, v_ref, kb_ref, vb_ref):
    k = k_ref[...]
    v = v_ref[...]
    zpad = jnp.zeros((HD, S - T), jnp.float32)
    for rb in range(_RPS):
        kb_ref[rb, :, T:] = zpad
        vb_ref[rb, :, T:] = zpad
        for h in range(H):
            kb_ref[rb, pl.ds(h * D, D), :T] = jnp.transpose(k[0, rb, :, h, :])
            vb_ref[rb, pl.ds(h * D, D), :T] = jnp.transpose(v[0, rb, :, h, :])


def _tc_buf(keys, values):
    bpg = B // _RPS
    in5 = pl.BlockSpec((1, _RPS, T, H, D),
                       lambda i: (i // bpg, i % bpg, 0, 0, 0))
    buf = pl.BlockSpec((_RPS, HD, S), lambda i: (i, 0, 0))
    return pl.pallas_call(
        _tc_buf_body,
        grid=(LB // _RPS,),
        in_specs=[in5, in5],
        out_specs=[buf, buf],
        out_shape=[
            jax.ShapeDtypeStruct((LB, HD, S), jnp.float32),
            jax.ShapeDtypeStruct((LB, HD, S), jnp.float32),
        ],
    )(keys, values)


def kernel(keys, values, keys_buf, values_buf):
    ko, vo = _sc_copy(keys, values)
    kbp, vbp = _tc_buf(keys, values)
    kb = jnp.transpose(kbp.reshape(L, B, H, D, S), (0, 1, 4, 2, 3))
    vb = jnp.transpose(vbp.reshape(L, B, H, D, S), (0, 1, 4, 2, 3))
    return (kb, vb, ko, vo)


# speedup vs baseline: 1.9265x; 1.0261x over previous
import jax
import jax.numpy as jnp
from jax import lax
from jax.experimental import pallas as pl
from jax.experimental.pallas import tpu as pltpu
from jax.experimental.pallas import tpu_sc as plsc

L, B, T, H, D = 8, 8, 64, 8, 64
S = 512
LB = L * B
HD = H * D
NC, NS = 2, 16
NW = NC * NS
ROWS_PER_W = LB // NW

_SC_PARAMS = pltpu.CompilerParams(use_tc_tiling_on_sc=True)
_MESH = plsc.VectorSubcoreMesh(core_axis_name="c", subcore_axis_name="s")


def _sc_copy_body(k_hbm, v_hbm, ko_hbm, vo_hbm, stage, sem):
    c = lax.axis_index("c")
    s = lax.axis_index("s")
    wid = s * NC + c
    for rl in range(ROWS_PER_W):
        r = wid * ROWS_PER_W + rl
        li = r // B
        bi = r % B
        pltpu.async_copy(k_hbm.at[li, bi], stage, sem).wait()
        pltpu.async_copy(stage, ko_hbm.at[li, bi], sem).wait()
        pltpu.async_copy(v_hbm.at[li, bi], stage, sem).wait()
        pltpu.async_copy(stage, vo_hbm.at[li, bi], sem).wait()


_sc_copy = pl.kernel(
    _sc_copy_body,
    out_type=[
        jax.ShapeDtypeStruct((L, B, T, H, D), jnp.float32),
        jax.ShapeDtypeStruct((L, B, T, H, D), jnp.float32),
    ],
    mesh=_MESH,
    scratch_types=[
        pltpu.VMEM((T, H, D), jnp.float32),
        pltpu.SemaphoreType.DMA,
    ],
    compiler_params=_SC_PARAMS,
)


_RPS = 8


def _tc_buf_body(k_ref, v_ref, kb_ref, vb_ref):
    k = k_ref[...]
    v = v_ref[...]
    zpad = jnp.zeros((HD, S - T), jnp.float32)
    for rb in range(_RPS):
        kb_ref[rb, :, T:] = zpad
        vb_ref[rb, :, T:] = zpad
        for h in range(H):
            kb_ref[rb, pl.ds(h * D, D), :T] = jnp.transpose(k[0, rb, :, h, :])
            vb_ref[rb, pl.ds(h * D, D), :T] = jnp.transpose(v[0, rb, :, h, :])


def _tc_buf(keys, values):
    bpg = B // _RPS
    in5 = pl.BlockSpec((1, _RPS, T, H, D),
                       lambda i: (i // bpg, i % bpg, 0, 0, 0))
    buf = pl.BlockSpec((_RPS, HD, S), lambda i: (i, 0, 0))
    return pl.pallas_call(
        _tc_buf_body,
        grid=(LB // _RPS,),
        in_specs=[in5, in5],
        out_specs=[buf, buf],
        out_shape=[
            jax.ShapeDtypeStruct((LB, HD, S), jnp.float32),
            jax.ShapeDtypeStruct((LB, HD, S), jnp.float32),
        ],
    )(keys, values)


def kernel(keys, values, keys_buf, values_buf):
    ko, vo = _sc_copy(keys, values)
    kbp, vbp = _tc_buf(keys, values)
    kb = jnp.transpose(kbp.reshape(L, B, H, D, S), (0, 1, 4, 2, 3))
    vb = jnp.transpose(vbp.reshape(L, B, H, D, S), (0, 1, 4, 2, 3))
    return (kb, vb, ko, vo)
